# baseline (speedup 1.0000x reference)
import functools

import jax
import jax.numpy as jnp
from jax import lax
from jax.experimental import pallas as pl
from jax.experimental.pallas import tpu as pltpu
from jax.experimental.pallas import tpu_sc as plsc

WINDOW = 512
STRIDE = 256
T = 32768
D = 256
OSZ = (T - WINDOW) // STRIDE + 1


def _make_sc_kernel():
  info = plsc.get_sparse_core_info()
  nc, ns = info.num_cores, info.num_subcores
  nw = nc * ns
  wpw = OSZ // nw + 1

  mesh = plsc.VectorSubcoreMesh(core_axis_name="c", subcore_axis_name="s")

  @functools.partial(
      pl.kernel,
      mesh=mesh,
      out_type=jax.ShapeDtypeStruct((OSZ * WINDOW, D), jnp.float32),
      scratch_types=[
          pltpu.SemaphoreType.DMA,
      ],
  )
  def body(in_hbm, out_hbm, sem):
    wid = lax.axis_index("s") * nc + lax.axis_index("c")

    def win_copy(i):
      return pltpu.make_async_copy(
          in_hbm.at[pl.ds(i * STRIDE, WINDOW)],
          out_hbm.at[pl.ds(i * WINDOW, WINDOW)],
          sem)

    for k in range(wpw):
      i = wid * wpw + k

      @pl.when(i <= OSZ - 1)
      def _():
        win_copy(i).start()

    for k in range(wpw):
      i = wid * wpw + k

      @pl.when(i <= OSZ - 1)
      def _():
        win_copy(i).wait()

  return body


_sc_unfold = _make_sc_kernel()


def kernel(input):
  flat = _sc_unfold(input)
  return flat.reshape(OSZ, WINDOW, D)


# speedup vs baseline: 37.1178x; 37.1178x over previous
import functools

import jax
import jax.numpy as jnp
from jax import lax
from jax.experimental import pallas as pl
from jax.experimental.pallas import tpu as pltpu
from jax.experimental.pallas import tpu_sc as plsc

WINDOW = 512
STRIDE = 256
T = 32768
D = 256
OSZ = (T - WINDOW) // STRIDE + 1
NBLK = T // STRIDE
CHUNK = 128
CPB = STRIDE // CHUNK


def _make_sc_kernel():
  info = plsc.get_sparse_core_info()
  nc, ns = info.num_cores, info.num_subcores
  nw = nc * ns
  bpw = NBLK // nw
  nchunks = bpw * CPB

  mesh = plsc.VectorSubcoreMesh(core_axis_name="c", subcore_axis_name="s")

  @functools.partial(
      pl.kernel,
      mesh=mesh,
      out_type=jax.ShapeDtypeStruct((OSZ * WINDOW, D), jnp.float32),
      scratch_types=[
          pltpu.VMEM((CHUNK, D), jnp.float32),
          pltpu.VMEM((CHUNK, D), jnp.float32),
          pltpu.SemaphoreType.DMA,
          pltpu.SemaphoreType.DMA,
          pltpu.SemaphoreType.DMA,
          pltpu.SemaphoreType.DMA,
          pltpu.SemaphoreType.DMA,
          pltpu.SemaphoreType.DMA,
      ],
  )
  def body(in_hbm, out_hbm, buf0, buf1,
           in_s0, in_s1, a_s0, a_s1, b_s0, b_s1):
    wid = lax.axis_index("s") * nc + lax.axis_index("c")
    bufs = (buf0, buf1)
    in_sems = (in_s0, in_s1)
    a_sems = (a_s0, a_s1)
    b_sems = (b_s0, b_s1)

    def blk(c):
      return wid * bpw + c // CPB

    def in_copy(c, x):
      src = blk(c) * STRIDE + (c % CPB) * CHUNK
      return pltpu.make_async_copy(
          in_hbm.at[pl.ds(src, CHUNK)], bufs[x], in_sems[x])

    def out_a(c, x):
      dst = blk(c) * WINDOW + (c % CPB) * CHUNK
      return pltpu.make_async_copy(
          bufs[x], out_hbm.at[pl.ds(dst, CHUNK)], a_sems[x])

    def out_b(c, x):
      dst = (blk(c) - 1) * WINDOW + STRIDE + (c % CPB) * CHUNK
      return pltpu.make_async_copy(
          bufs[x], out_hbm.at[pl.ds(dst, CHUNK)], b_sems[x])

    in_copy(0, 0).start()
    for c in range(nchunks):
      x = c % 2
      y = (c + 1) % 2
      if c + 1 < nchunks:
        if c - 1 >= 0:
          @pl.when(blk(c - 1) <= OSZ - 1)
          def _():
            out_a(c - 1, y).wait()

          @pl.when(blk(c - 1) >= 1)
          def _():
            out_b(c - 1, y).wait()

        in_copy(c + 1, y).start()

      in_copy(c, x).wait()

      @pl.when(blk(c) <= OSZ - 1)
      def _():
        out_a(c, x).start()

      @pl.when(blk(c) >= 1)
      def _():
        out_b(c, x).start()

    for c in (nchunks - 2, nchunks - 1):
      x = c % 2

      @pl.when(blk(c) <= OSZ - 1)
      def _():
        out_a(c, x).wait()

      @pl.when(blk(c) >= 1)
      def _():
        out_b(c, x).wait()

  return body


_sc_unfold = _make_sc_kernel()


def kernel(input):
  flat = _sc_unfold(input)
  return flat.reshape(OSZ, WINDOW, D)
